# baseline (speedup 1.0000x reference)
import functools

import jax
import jax.numpy as jnp
from jax import lax
from jax.experimental import pallas as pl
from jax.experimental.pallas import tpu as pltpu
from jax.experimental.pallas import tpu_sc as plsc

B = 16
N = 65536
FEAT = 64
INV_T = 1.0 / 0.07

NUM_WORKERS = 32
CHUNK = 8192
N_CHUNKS = N // CHUNK
HROWS = N // 256
CROWS = CHUNK // 256


def _qt_kernel(f_ref, bt_ref, out_ref):
    f = f_ref[...] * INV_T
    dims = (((1,), (0,)), ((), ()))
    blk = bt_ref.shape[1]
    q = lax.dot_general(f, bt_ref[...], dims,
                        preferred_element_type=jnp.float32)
    out_ref[...] = q.reshape(32, blk // 128, 128)


def _compute_qt(f_pair, bank_t):
    blk = 4096
    grid = (N // blk,)
    return pl.pallas_call(
        _qt_kernel,
        grid=grid,
        in_specs=[
            pl.BlockSpec((32, FEAT), lambda i: (0, 0)),
            pl.BlockSpec((FEAT, blk), lambda i: (0, i)),
        ],
        out_specs=pl.BlockSpec((32, blk // 128, 128), lambda i: (0, i, 0)),
        out_shape=jax.ShapeDtypeStruct((32, N // 128, 128), jnp.float32),
    )(f_pair, bank_t)


def _sc_body(qt_hbm, idx_hbm, oa, ob,
             q_v, idx_v, oa_v, ob_v, sem_a0, sem_a1, sem_b0, sem_b1):
    c = lax.axis_index("c")
    s = lax.axis_index("s")
    wid = s * 2 + c
    b = lax.rem(wid, B)
    hi = wid >= B
    a_sems = (sem_a0, sem_a1)
    b_sems = (sem_b0, sem_b1)

    def out_dma(ch, bf, drain):
        col_a = pl.ds(ch * CROWS, CROWS)
        col_b = pl.ds(N // 256 + ch * CROWS, CROWS)

        @pl.when(jnp.logical_not(hi))
        def _():
            da = pltpu.make_async_copy(oa_v.at[bf], oa.at[b, col_a],
                                       a_sems[bf])
            db = pltpu.make_async_copy(ob_v.at[bf], oa.at[b, col_b],
                                       b_sems[bf])
            if drain:
                da.wait()
                db.wait()
            else:
                da.start()
                db.start()

        @pl.when(hi)
        def _():
            da = pltpu.make_async_copy(oa_v.at[bf], ob.at[b, col_a],
                                       a_sems[bf])
            db = pltpu.make_async_copy(ob_v.at[bf], ob.at[b, col_b],
                                       b_sems[bf])
            if drain:
                da.wait()
                db.wait()
            else:
                da.start()
                db.start()

    pltpu.sync_copy(idx_hbm.at[b], idx_v)
    pltpu.sync_copy(qt_hbm.at[wid], q_v)

    out_pending = [None, None]
    for ch in range(N_CHUNKS):
        bf = ch % 2
        if out_pending[bf] is not None:
            out_dma(out_pending[bf], bf, True)

        @plsc.parallel_loop(ch * CROWS, (ch + 1) * CROWS, 1, unroll=2)
        def _gather(p):
            lp = p - ch * CROWS
            for jj in range(0, 128, 16):
                w = idx_v[p, pl.ds(jj, 16)]
                ia = lax.bitwise_and(w, 0xFFFF)
                ib = lax.shift_right_logical(w, 16)
                oa_v[bf, lp, pl.ds(jj, 16)] = plsc.load_gather(
                    q_v, [lax.shift_right_logical(ia, 7),
                          lax.bitwise_and(ia, 127)])
                ob_v[bf, lp, pl.ds(jj, 16)] = plsc.load_gather(
                    q_v, [lax.shift_right_logical(ib, 7),
                          lax.bitwise_and(ib, 127)])

        out_dma(ch, bf, False)
        out_pending[bf] = ch
    for bf in range(2):
        if out_pending[bf] is not None:
            out_dma(out_pending[bf], bf, True)


def _sc_gather(qt, idxp):
    mesh = plsc.VectorSubcoreMesh(core_axis_name="c", subcore_axis_name="s")
    out_t = jax.ShapeDtypeStruct((B, N // 128, 128), jnp.float32)
    fn = functools.partial(
        pl.kernel,
        mesh=mesh,
        out_type=(out_t, out_t),
        scratch_types=[
            pltpu.VMEM((N // 128, 128), jnp.float32),
            pltpu.VMEM((HROWS, 128), jnp.int32),
            pltpu.VMEM((2, CROWS, 128), jnp.float32),
            pltpu.VMEM((2, CROWS, 128), jnp.float32),
            pltpu.SemaphoreType.DMA,
            pltpu.SemaphoreType.DMA,
            pltpu.SemaphoreType.DMA,
            pltpu.SemaphoreType.DMA,
        ],
        compiler_params=pltpu.CompilerParams(needs_layout_passes=False),
    )(_sc_body)
    return fn(qt, idxp)


def kernel(fea_f, fea_fenzi, fea_fenmu, y, idx, memory_fringe, memory_fenzi,
           memory_fenmu):
    del y
    idx16 = idx.astype(jnp.uint16)
    idxp = lax.bitcast_convert_type(
        jnp.stack([idx16[:, :N // 2], idx16[:, N // 2:]], axis=-1),
        jnp.int32)
    idxp = idxp.reshape(B, N // 256, 128)

    f_z = jnp.concatenate([fea_f, fea_fenmu], axis=0)
    f_m = jnp.concatenate([fea_f, fea_fenzi], axis=0)
    f_r = jnp.concatenate([fea_fenzi, fea_fenmu], axis=0)

    qt_z = _compute_qt(f_z, memory_fenzi.T)
    f_fenzi, fenmu_fenzi = _sc_gather(qt_z, idxp)
    qt_m = _compute_qt(f_m, memory_fenmu.T)
    f_fenmu, fenzi_fenmu = _sc_gather(qt_m, idxp)
    qt_r = _compute_qt(f_r, memory_fringe.T)
    fenzi_f, fenmu_f = _sc_gather(qt_r, idxp)

    return tuple(o.reshape(B, N, 1) for o in
                 (f_fenzi, f_fenmu, fenzi_f, fenzi_fenmu, fenmu_f,
                  fenmu_fenzi))


# speedup vs baseline: 1.0431x; 1.0431x over previous
import functools

import jax
import jax.numpy as jnp
from jax import lax
from jax.experimental import pallas as pl
from jax.experimental.pallas import tpu as pltpu
from jax.experimental.pallas import tpu_sc as plsc

B = 16
N = 65536
FEAT = 64
INV_T = 1.0 / 0.07

NUM_WORKERS = 32
CHUNK = 8192
N_CHUNKS = N // CHUNK
HROWS = N // 256
CROWS = CHUNK // 256


def _qt_kernel(f_ref, bt_ref, out_ref):
    f = f_ref[...] * INV_T
    dims = (((1,), (0,)), ((), ()))
    blk = bt_ref.shape[1]
    q = lax.dot_general(f, bt_ref[...], dims,
                        preferred_element_type=jnp.float32)
    out_ref[...] = q.reshape(32, blk // 128, 128)


def _compute_qt(f_pair, bank_t):
    blk = 4096
    grid = (N // blk,)
    return pl.pallas_call(
        _qt_kernel,
        grid=grid,
        in_specs=[
            pl.BlockSpec((32, FEAT), lambda i: (0, 0)),
            pl.BlockSpec((FEAT, blk), lambda i: (0, i)),
        ],
        out_specs=pl.BlockSpec((32, blk // 128, 128), lambda i: (0, i, 0)),
        out_shape=jax.ShapeDtypeStruct((32, N // 128, 128), jnp.float32),
    )(f_pair, bank_t)


def _qt2_kernel(fa_ref, fb_ref, bat_ref, bbt_ref, oa_ref, ob_ref):
    dims = (((1,), (0,)), ((), ()))
    blk = bat_ref.shape[1]
    fa = fa_ref[...] * INV_T
    fb = fb_ref[...] * INV_T
    qa = lax.dot_general(fa, bat_ref[...], dims,
                         preferred_element_type=jnp.float32)
    qb = lax.dot_general(fb, bbt_ref[...], dims,
                         preferred_element_type=jnp.float32)
    oa_ref[...] = qa.reshape(32, blk // 128, 128)
    ob_ref[...] = qb.reshape(32, blk // 128, 128)


def _compute_qt2(f_a, f_b, bank_at, bank_bt):
    blk = 4096
    grid = (N // blk,)
    out_t = jax.ShapeDtypeStruct((32, N // 128, 128), jnp.float32)
    return pl.pallas_call(
        _qt2_kernel,
        grid=grid,
        in_specs=[
            pl.BlockSpec((32, FEAT), lambda i: (0, 0)),
            pl.BlockSpec((32, FEAT), lambda i: (0, 0)),
            pl.BlockSpec((FEAT, blk), lambda i: (0, i)),
            pl.BlockSpec((FEAT, blk), lambda i: (0, i)),
        ],
        out_specs=[
            pl.BlockSpec((32, blk // 128, 128), lambda i: (0, i, 0)),
            pl.BlockSpec((32, blk // 128, 128), lambda i: (0, i, 0)),
        ],
        out_shape=(out_t, out_t),
    )(f_a, f_b, bank_at, bank_bt)


def _make_sc_body(nbanks):
    def body(*args):
        qts = args[:nbanks]
        idx_hbm = args[nbanks]
        outs = args[nbanks + 1:nbanks + 1 + 2 * nbanks]
        (q_v, idx_v, oa_v, ob_v,
         sem_a0, sem_a1, sem_b0, sem_b1) = args[nbanks + 1 + 2 * nbanks:]
        c = lax.axis_index("c")
        s = lax.axis_index("s")
        wid = s * 2 + c
        b = lax.rem(wid, B)
        hi = wid >= B
        a_sems = (sem_a0, sem_a1)
        b_sems = (sem_b0, sem_b1)

        def out_dma(t, ch, bf, drain):
            col_a = pl.ds(ch * CROWS, CROWS)
            col_b = pl.ds(N // 256 + ch * CROWS, CROWS)
            oa, ob = outs[2 * t], outs[2 * t + 1]

            @pl.when(jnp.logical_not(hi))
            def _():
                da = pltpu.make_async_copy(oa_v.at[bf], oa.at[b, col_a],
                                           a_sems[bf])
                db = pltpu.make_async_copy(ob_v.at[bf], oa.at[b, col_b],
                                           b_sems[bf])
                if drain:
                    da.wait()
                    db.wait()
                else:
                    da.start()
                    db.start()

            @pl.when(hi)
            def _():
                da = pltpu.make_async_copy(oa_v.at[bf], ob.at[b, col_a],
                                           a_sems[bf])
                db = pltpu.make_async_copy(ob_v.at[bf], ob.at[b, col_b],
                                           b_sems[bf])
                if drain:
                    da.wait()
                    db.wait()
                else:
                    da.start()
                    db.start()

        pltpu.sync_copy(idx_hbm.at[b], idx_v)

        out_pending = [None, None]
        for t in range(nbanks):
            pltpu.sync_copy(qts[t].at[wid], q_v)
            for ch in range(N_CHUNKS):
                bf = ch % 2
                if out_pending[bf] is not None:
                    out_dma(*out_pending[bf], bf, True)

                @plsc.parallel_loop(ch * CROWS, (ch + 1) * CROWS, 1,
                                    unroll=2)
                def _gather(p):
                    lp = p - ch * CROWS
                    for jj in range(0, 128, 16):
                        w = idx_v[p, pl.ds(jj, 16)]
                        ia = lax.bitwise_and(w, 0xFFFF)
                        ib = lax.shift_right_logical(w, 16)
                        oa_v[bf, lp, pl.ds(jj, 16)] = plsc.load_gather(
                            q_v, [lax.shift_right_logical(ia, 7),
                                  lax.bitwise_and(ia, 127)])
                        ob_v[bf, lp, pl.ds(jj, 16)] = plsc.load_gather(
                            q_v, [lax.shift_right_logical(ib, 7),
                                  lax.bitwise_and(ib, 127)])

                out_dma(t, ch, bf, False)
                out_pending[bf] = (t, ch)
        for bf in range(2):
            if out_pending[bf] is not None:
                out_dma(*out_pending[bf], bf, True)

    return body


def _sc_gather(qts, idxp):
    nbanks = len(qts)
    mesh = plsc.VectorSubcoreMesh(core_axis_name="c", subcore_axis_name="s")
    out_t = jax.ShapeDtypeStruct((B, N // 128, 128), jnp.float32)
    fn = functools.partial(
        pl.kernel,
        mesh=mesh,
        out_type=(out_t,) * (2 * nbanks),
        scratch_types=[
            pltpu.VMEM((N // 128, 128), jnp.float32),
            pltpu.VMEM((HROWS, 128), jnp.int32),
            pltpu.VMEM((2, CROWS, 128), jnp.float32),
            pltpu.VMEM((2, CROWS, 128), jnp.float32),
            pltpu.SemaphoreType.DMA,
            pltpu.SemaphoreType.DMA,
            pltpu.SemaphoreType.DMA,
            pltpu.SemaphoreType.DMA,
        ],
        compiler_params=pltpu.CompilerParams(needs_layout_passes=False),
    )(_make_sc_body(nbanks))
    return fn(*qts, idxp)


def kernel(fea_f, fea_fenzi, fea_fenmu, y, idx, memory_fringe, memory_fenzi,
           memory_fenmu):
    del y
    idx16 = idx.astype(jnp.uint16)
    idxp = lax.bitcast_convert_type(
        jnp.stack([idx16[:, :N // 2], idx16[:, N // 2:]], axis=-1),
        jnp.int32)
    idxp = idxp.reshape(B, N // 256, 128)

    f_z = jnp.concatenate([fea_f, fea_fenmu], axis=0)
    f_m = jnp.concatenate([fea_f, fea_fenzi], axis=0)
    f_r = jnp.concatenate([fea_fenzi, fea_fenmu], axis=0)

    qt_z = _compute_qt(f_z, memory_fenzi.T)
    f_fenzi, fenmu_fenzi = _sc_gather([qt_z], idxp)
    qt_m, qt_r = _compute_qt2(f_m, f_r, memory_fenmu.T, memory_fringe.T)
    f_fenmu, fenzi_fenmu, fenzi_f, fenmu_f = _sc_gather([qt_m, qt_r], idxp)

    return tuple(o.reshape(B, N, 1) for o in
                 (f_fenzi, f_fenmu, fenzi_f, fenzi_fenmu, fenmu_f,
                  fenmu_fenzi))


# speedup vs baseline: 1.1257x; 1.0792x over previous
import functools

import jax
import jax.numpy as jnp
from jax import lax
from jax.experimental import pallas as pl
from jax.experimental.pallas import tpu as pltpu
from jax.experimental.pallas import tpu_sc as plsc

B = 16
N = 65536
FEAT = 64
INV_T = 1.0 / 0.07

NUM_WORKERS = 32
ROWS = 6 * B
ROUNDS = ROWS // NUM_WORKERS
CHUNK = 8192
N_CHUNKS = N // CHUNK


def _qt_kernel(fz_ref, fm_ref, fr_ref, bz_ref, bm_ref, br_ref, out_ref):
    fz = fz_ref[...] * INV_T
    fm = fm_ref[...] * INV_T
    fr = fr_ref[...] * INV_T
    dims = (((1,), (0,)), ((), ()))
    blk = bz_ref.shape[1]
    qz = lax.dot_general(
        fz, bz_ref[...], dims, preferred_element_type=jnp.float32)
    qm = lax.dot_general(
        fm, bm_ref[...], dims, preferred_element_type=jnp.float32)
    qr = lax.dot_general(
        fr, br_ref[...], dims, preferred_element_type=jnp.float32)
    out_ref[0:32] = qz.reshape(32, blk // 128, 128)
    out_ref[32:64] = qm.reshape(32, blk // 128, 128)
    out_ref[64:96] = qr.reshape(32, blk // 128, 128)


def _compute_qt(f_z, f_m, f_r, bank_zt, bank_mt, bank_rt):
    blk = 4096
    grid = (N // blk,)
    return pl.pallas_call(
        _qt_kernel,
        grid=grid,
        in_specs=[
            pl.BlockSpec((32, FEAT), lambda i: (0, 0)),
            pl.BlockSpec((32, FEAT), lambda i: (0, 0)),
            pl.BlockSpec((32, FEAT), lambda i: (0, 0)),
            pl.BlockSpec((FEAT, blk), lambda i: (0, i)),
            pl.BlockSpec((FEAT, blk), lambda i: (0, i)),
            pl.BlockSpec((FEAT, blk), lambda i: (0, i)),
        ],
        out_specs=pl.BlockSpec((ROWS, blk // 128, 128), lambda i: (0, i, 0)),
        out_shape=jax.ShapeDtypeStruct((ROWS, N // 128, 128), jnp.float32),
    )(f_z, f_m, f_r, bank_zt, bank_mt, bank_rt)


HROWS = N // 256
CROWS = CHUNK // 256


def _sc_body(qt_hbm, idx_hbm, o0, o1, o2, o3, o4, o5,
             q_v, idx_v, oa_v, ob_v, sem_a0, sem_a1, sem_b0, sem_b1):
    c = lax.axis_index("c")
    s = lax.axis_index("s")
    wid = s * 2 + c
    b = lax.rem(wid, B)
    hi = wid >= B
    outs_lo = (o0, o1, o2)
    outs_hi = (o5, o3, o4)
    a_sems = (sem_a0, sem_a1)
    b_sems = (sem_b0, sem_b1)

    def out_dma(t, ch, bf, make_only):
        col_a = pl.ds(ch * CROWS, CROWS)
        col_b = pl.ds(N // 256 + ch * CROWS, CROWS)

        @pl.when(jnp.logical_not(hi))
        def _():
            da = pltpu.make_async_copy(oa_v.at[bf], outs_lo[t].at[b, col_a],
                                       a_sems[bf])
            db = pltpu.make_async_copy(ob_v.at[bf], outs_lo[t].at[b, col_b],
                                       b_sems[bf])
            if make_only:
                da.wait()
                db.wait()
            else:
                da.start()
                db.start()

        @pl.when(hi)
        def _():
            da = pltpu.make_async_copy(oa_v.at[bf], outs_hi[t].at[b, col_a],
                                       a_sems[bf])
            db = pltpu.make_async_copy(ob_v.at[bf], outs_hi[t].at[b, col_b],
                                       b_sems[bf])
            if make_only:
                da.wait()
                db.wait()
            else:
                da.start()
                db.start()

    pltpu.sync_copy(idx_hbm.at[b], idx_v)

    out_pending = [None, None]
    for t in range(ROUNDS):
        r = t * NUM_WORKERS + wid
        pltpu.sync_copy(qt_hbm.at[r], q_v)
        for ch in range(N_CHUNKS):
            bf = ch % 2
            if out_pending[bf] is not None:
                out_dma(*out_pending[bf], bf, True)

            @plsc.parallel_loop(ch * CROWS, (ch + 1) * CROWS, 1, unroll=2)
            def _gather(p):
                lp = p - ch * CROWS
                for jj in range(0, 128, 16):
                    w = idx_v[p, pl.ds(jj, 16)]
                    ia = lax.bitwise_and(w, 0xFFFF)
                    ib = lax.shift_right_logical(w, 16)
                    oa_v[bf, lp, pl.ds(jj, 16)] = plsc.load_gather(
                        q_v, [lax.shift_right_logical(ia, 7),
                              lax.bitwise_and(ia, 127)])
                    ob_v[bf, lp, pl.ds(jj, 16)] = plsc.load_gather(
                        q_v, [lax.shift_right_logical(ib, 7),
                              lax.bitwise_and(ib, 127)])

            out_dma(t, ch, bf, False)
            out_pending[bf] = (t, ch)
    for bf in range(2):
        if out_pending[bf] is not None:
            out_dma(*out_pending[bf], bf, True)


def _sc_gather(qt, idxp):
    mesh = plsc.VectorSubcoreMesh(core_axis_name="c", subcore_axis_name="s")
    out_t = jax.ShapeDtypeStruct((B, N // 128, 128), jnp.float32)
    fn = functools.partial(
        pl.kernel,
        mesh=mesh,
        out_type=(out_t,) * 6,
        scratch_types=[
            pltpu.VMEM((N // 128, 128), jnp.float32),
            pltpu.VMEM((HROWS, 128), jnp.int32),
            pltpu.VMEM((2, CROWS, 128), jnp.float32),
            pltpu.VMEM((2, CROWS, 128), jnp.float32),
            pltpu.SemaphoreType.DMA,
            pltpu.SemaphoreType.DMA,
            pltpu.SemaphoreType.DMA,
            pltpu.SemaphoreType.DMA,
        ],
        compiler_params=pltpu.CompilerParams(needs_layout_passes=False),
    )(_sc_body)
    return fn(qt, idxp)


def kernel(fea_f, fea_fenzi, fea_fenmu, y, idx, memory_fringe, memory_fenzi,
           memory_fenmu):
    del y
    idx16 = idx.astype(jnp.uint16)
    idxp = lax.bitcast_convert_type(
        jnp.stack([idx16[:, :N // 2], idx16[:, N // 2:]], axis=-1),
        jnp.int32)
    idxp = idxp.reshape(B, N // 256, 128)
    f_z = jnp.concatenate([fea_f, fea_fenmu], axis=0)
    f_m = jnp.concatenate([fea_f, fea_fenzi], axis=0)
    f_r = jnp.concatenate([fea_fenzi, fea_fenmu], axis=0)

    qt = _compute_qt(f_z, f_m, f_r, memory_fenzi.T, memory_fenmu.T,
                     memory_fringe.T)
    outs = _sc_gather(qt, idxp)
    return tuple(o.reshape(B, N, 1) for o in outs)
